# baseline (speedup 1.0000x reference)
import numpy as np
import jax
import jax.numpy as jnp
from jax.experimental import pallas as pl
from jax.experimental.pallas import tpu as pltpu

SEQ = 2048
NF = SEQ // 2
HID = 4096
NPS = 88
HBLK = 1024

_l = np.arange(SEQ, dtype=np.float64)[:, None]
_k = np.arange(1, NF + 1, dtype=np.float64)[None, :]
_ang = (2.0 * np.pi / SEQ) * _l * _k
_D2_NP = np.concatenate(
    [np.cos(_ang), -np.sin(_ang)], axis=1
).astype(np.float32) / np.sqrt(SEQ).astype(np.float32)


def _reduce_body(x_ref, sw_ref, o_ref):
    o_ref[...] = jnp.dot(sw_ref[...], x_ref[...],
                         preferred_element_type=jnp.float32)[None]


def _dft_body(r_ref, d2_ref, o_ref):
    b, ln = r_ref.shape
    r3 = r_ref[...].reshape(b, SEQ, ln // SEQ)
    m = jnp.sum(r3, axis=-1) * (SEQ / ln)
    o_ref[...] = jnp.dot(m, d2_ref[...], preferred_element_type=jnp.float32)


def _mlp_body(xri_ref, w1_ref, b1_ref, w2_ref, b2_ref, wg_ref, wn_ref,
              tr_ref, eps_ref, o_ref, accr, acci):
    i = pl.program_id(0)
    nsteps = pl.num_programs(0)
    bsz = xri_ref.shape[0]

    xr = xri_ref[:, :NF]
    xi = xri_ref[:, NF:]
    a2 = jnp.concatenate([xr, xi], axis=0)
    b2m = jnp.concatenate([xi, xr], axis=0)

    y0 = jnp.dot(a2, w1_ref[0], preferred_element_type=jnp.float32)
    y1 = jnp.dot(b2m, w1_ref[1], preferred_element_type=jnp.float32)
    o1r = jax.nn.relu(y0[:bsz] - y1[:bsz] + b1_ref[0])
    o1i = jax.nn.relu(y0[bsz:] + y1[bsz:] + b1_ref[1])
    o2 = jnp.concatenate([o1r, o1i], axis=0)

    u = jnp.dot(o2, w2_ref[0], preferred_element_type=jnp.float32)
    v = jnp.dot(o2, w2_ref[1], preferred_element_type=jnp.float32)

    @pl.when(i == 0)
    def _init():
        accr[...] = jnp.zeros_like(accr)
        acci[...] = jnp.zeros_like(acci)

    accr[...] += u[:bsz] - v[bsz:]
    acci[...] += u[bsz:] + v[:bsz]

    @pl.when(i == nsteps - 1)
    def _fin():
        ar = accr[...] + b2_ref[0]
        ai = acci[...] + b2_ref[1]
        mag = jnp.sqrt(ar * ar + ai * ai)
        clean = jnp.dot(mag, wg_ref[...], preferred_element_type=jnp.float32)
        zn = jnp.dot(mag, wn_ref[...], preferred_element_type=jnp.float32)
        softplus = jnp.log1p(jnp.exp(-jnp.abs(zn))) + jnp.maximum(zn, 0.0)
        noisy = clean + softplus + eps_ref[0, 0]
        logits = jnp.where(tr_ref[0, 0] != 0.0, noisy, clean)

        col = jax.lax.broadcasted_iota(jnp.int32, logits.shape, 1)
        m1 = jnp.max(logits, axis=1, keepdims=True)
        i1 = jnp.min(jnp.where(logits == m1, col, NPS), axis=1, keepdims=True)
        l2 = jnp.where(col == i1, -jnp.inf, logits)
        m2 = jnp.max(l2, axis=1, keepdims=True)
        i2 = jnp.min(jnp.where(l2 == m2, col, NPS), axis=1, keepdims=True)
        e2 = jnp.exp(m2 - m1)
        denom = 1.0 + e2
        p1 = 1.0 / denom
        p2 = e2 / denom
        o_ref[...] = (jnp.where(col == i1, p1, 0.0)
                      + jnp.where(col == i2, p2, 0.0))


def kernel(x, start_w, start_b, w1, b1, w2, b2, w_gate, w_noise,
           training=False, noise_epsilon=0.01):
    B_, C_, L_, N_ = x.shape
    LN = L_ * N_
    x2 = x.reshape(B_ * C_, LN)

    r = pl.pallas_call(
        _reduce_body,
        grid=(B_,),
        in_specs=[
            pl.BlockSpec((C_, LN), lambda i: (i, 0)),
            pl.BlockSpec((1, C_), lambda i: (0, 0)),
        ],
        out_specs=pl.BlockSpec((1, 1, LN), lambda i: (i, 0, 0)),
        out_shape=jax.ShapeDtypeStruct((B_, 1, LN), jnp.float32),
    )(x2, start_w)
    r = r.reshape(B_, LN)

    d2 = jnp.asarray(_D2_NP)
    xri = pl.pallas_call(
        _dft_body,
        in_specs=[pl.BlockSpec((B_, LN), lambda: (0, 0)),
                  pl.BlockSpec((SEQ, 2 * NF), lambda: (0, 0))],
        out_specs=pl.BlockSpec((B_, 2 * NF), lambda: (0, 0)),
        out_shape=jax.ShapeDtypeStruct((B_, 2 * NF), jnp.float32),
    )(r, d2)

    tr = jnp.asarray(training, jnp.float32).reshape(1, 1)
    eps = jnp.asarray(noise_epsilon, jnp.float32).reshape(1, 1)
    nh = HID // HBLK

    gates = pl.pallas_call(
        _mlp_body,
        grid=(nh,),
        in_specs=[
            pl.BlockSpec((B_, 2 * NF), lambda i: (0, 0)),
            pl.BlockSpec((2, NF, HBLK), lambda i: (0, 0, i)),
            pl.BlockSpec((2, HBLK), lambda i: (0, i)),
            pl.BlockSpec((2, HBLK, NF), lambda i: (0, i, 0)),
            pl.BlockSpec((2, NF), lambda i: (0, 0)),
            pl.BlockSpec((NF, NPS), lambda i: (0, 0)),
            pl.BlockSpec((NF, NPS), lambda i: (0, 0)),
            pl.BlockSpec((1, 1), lambda i: (0, 0),
                         memory_space=pltpu.SMEM),
            pl.BlockSpec((1, 1), lambda i: (0, 0),
                         memory_space=pltpu.SMEM),
        ],
        out_specs=pl.BlockSpec((B_, NPS), lambda i: (0, 0)),
        out_shape=jax.ShapeDtypeStruct((B_, NPS), jnp.float32),
        scratch_shapes=[pltpu.VMEM((B_, NF), jnp.float32),
                        pltpu.VMEM((B_, NF), jnp.float32)],
    )(xri, w1, b1, w2, b2, w_gate, w_noise, tr, eps)
    return gates


# speedup vs baseline: 5.7799x; 5.7799x over previous
import numpy as np
import jax
import jax.numpy as jnp
from jax.experimental import pallas as pl
from jax.experimental.pallas import tpu as pltpu

SEQ = 2048
NF = SEQ // 2
HID = 4096
NPS = 88
HBLK = 1024

_l = np.arange(SEQ, dtype=np.float64)[:, None]
_k = np.arange(1, NF + 1, dtype=np.float64)[None, :]
_ang = (2.0 * np.pi / SEQ) * _l * _k
_D2_NP = (np.concatenate([np.cos(_ang), -np.sin(_ang)], axis=1)
          / np.sqrt(SEQ)).astype(np.float32)


def _front_body(xt_ref, wext_ref, d2_ref, o_ref):
    xs = jnp.dot(wext_ref[...], xt_ref[...],
                 preferred_element_type=jnp.float32)
    o_ref[...] = jnp.dot(xs, d2_ref[...],
                         preferred_element_type=jnp.float32)[None]


def _mlp_body(xri_ref, w1_ref, b1_ref, w2_ref, b2_ref, wg_ref, wn_ref,
              tr_ref, eps_ref, o_ref, accr, acci):
    i = pl.program_id(0)
    nsteps = pl.num_programs(0)
    bsz = xri_ref.shape[0]

    xr = xri_ref[:, :NF]
    xi = xri_ref[:, NF:]
    a2 = jnp.concatenate([xr, xi], axis=0)
    b2m = jnp.concatenate([xi, xr], axis=0)

    y0 = jnp.dot(a2, w1_ref[0], preferred_element_type=jnp.float32)
    y1 = jnp.dot(b2m, w1_ref[1], preferred_element_type=jnp.float32)
    o1r = jax.nn.relu(y0[:bsz] - y1[:bsz] + b1_ref[0])
    o1i = jax.nn.relu(y0[bsz:] + y1[bsz:] + b1_ref[1])
    o2 = jnp.concatenate([o1r, o1i], axis=0)

    u = jnp.dot(o2, w2_ref[0], preferred_element_type=jnp.float32)
    v = jnp.dot(o2, w2_ref[1], preferred_element_type=jnp.float32)

    @pl.when(i == 0)
    def _init():
        accr[...] = jnp.zeros_like(accr)
        acci[...] = jnp.zeros_like(acci)

    accr[...] += u[:bsz] - v[bsz:]
    acci[...] += u[bsz:] + v[:bsz]

    @pl.when(i == nsteps - 1)
    def _fin():
        ar = accr[...] + b2_ref[0]
        ai = acci[...] + b2_ref[1]
        mag = jnp.sqrt(ar * ar + ai * ai)
        clean = jnp.dot(mag, wg_ref[...], preferred_element_type=jnp.float32)
        zn = jnp.dot(mag, wn_ref[...], preferred_element_type=jnp.float32)
        softplus = jnp.log1p(jnp.exp(-jnp.abs(zn))) + jnp.maximum(zn, 0.0)
        noisy = clean + softplus + eps_ref[0, 0]
        logits = jnp.where(tr_ref[0, 0] != 0.0, noisy, clean)

        col = jax.lax.broadcasted_iota(jnp.int32, logits.shape, 1)
        m1 = jnp.max(logits, axis=1, keepdims=True)
        i1 = jnp.min(jnp.where(logits == m1, col, NPS), axis=1, keepdims=True)
        l2 = jnp.where(col == i1, -jnp.inf, logits)
        m2 = jnp.max(l2, axis=1, keepdims=True)
        i2 = jnp.min(jnp.where(l2 == m2, col, NPS), axis=1, keepdims=True)
        e2 = jnp.exp(m2 - m1)
        denom = 1.0 + e2
        p1 = 1.0 / denom
        p2 = e2 / denom
        o_ref[...] = (jnp.where(col == i1, p1, 0.0)
                      + jnp.where(col == i2, p2, 0.0))


def kernel(x, start_w, start_b, w1, b1, w2, b2, w_gate, w_noise,
           training=False, noise_epsilon=0.01):
    B_, C_, L_, N_ = x.shape
    CN = C_ * N_
    xt = x.transpose(0, 1, 3, 2).reshape(B_ * CN, L_)
    wext = jnp.repeat(start_w[0] / N_, N_).reshape(1, CN)

    d2 = jnp.asarray(_D2_NP)
    xri = pl.pallas_call(
        _front_body,
        grid=(B_,),
        in_specs=[
            pl.BlockSpec((CN, L_), lambda i: (i, 0)),
            pl.BlockSpec((1, CN), lambda i: (0, 0)),
            pl.BlockSpec((SEQ, 2 * NF), lambda i: (0, 0)),
        ],
        out_specs=pl.BlockSpec((1, 1, 2 * NF), lambda i: (i, 0, 0)),
        out_shape=jax.ShapeDtypeStruct((B_, 1, 2 * NF), jnp.float32),
    )(xt, wext, d2)
    xri = xri.reshape(B_, 2 * NF)

    tr = jnp.asarray(training, jnp.float32).reshape(1, 1)
    eps = jnp.asarray(noise_epsilon, jnp.float32).reshape(1, 1)
    nh = HID // HBLK

    gates = pl.pallas_call(
        _mlp_body,
        grid=(nh,),
        in_specs=[
            pl.BlockSpec((B_, 2 * NF), lambda i: (0, 0)),
            pl.BlockSpec((2, NF, HBLK), lambda i: (0, 0, i)),
            pl.BlockSpec((2, HBLK), lambda i: (0, i)),
            pl.BlockSpec((2, HBLK, NF), lambda i: (0, i, 0)),
            pl.BlockSpec((2, NF), lambda i: (0, 0)),
            pl.BlockSpec((NF, NPS), lambda i: (0, 0)),
            pl.BlockSpec((NF, NPS), lambda i: (0, 0)),
            pl.BlockSpec((1, 1), lambda i: (0, 0),
                         memory_space=pltpu.SMEM),
            pl.BlockSpec((1, 1), lambda i: (0, 0),
                         memory_space=pltpu.SMEM),
        ],
        out_specs=pl.BlockSpec((B_, NPS), lambda i: (0, 0)),
        out_shape=jax.ShapeDtypeStruct((B_, NPS), jnp.float32),
        scratch_shapes=[pltpu.VMEM((B_, NF), jnp.float32),
                        pltpu.VMEM((B_, NF), jnp.float32)],
    )(xri, w1, b1, w2, b2, w_gate, w_noise, tr, eps)
    return gates


# speedup vs baseline: 6.6476x; 1.1501x over previous
import numpy as np
import jax
import jax.numpy as jnp
from jax.experimental import pallas as pl
from jax.experimental.pallas import tpu as pltpu

SEQ = 2048
NF = SEQ // 2
HID = 4096
NPS = 88
HBLK = 1024

_l = np.arange(SEQ, dtype=np.float64)[:, None]
_k = np.arange(1, NF + 1, dtype=np.float64)[None, :]
_ang = (2.0 * np.pi / SEQ) * _l * _k
_D2_NP = (np.concatenate([np.cos(_ang), -np.sin(_ang)], axis=1)
          / np.sqrt(SEQ)).astype(np.float32)


def _front_body(xt_ref, wext_ref, o_ref):
    o_ref[...] = jnp.dot(wext_ref[...], xt_ref[...],
                         preferred_element_type=jnp.float32)[None]


def _dft_body(xs_ref, d2_ref, o_ref):
    o_ref[...] = jnp.dot(xs_ref[...], d2_ref[...],
                         preferred_element_type=jnp.float32)


def _mlp_body(xri_ref, w1_ref, b1_ref, w2_ref, b2_ref, wg_ref, wn_ref,
              tr_ref, eps_ref, o_ref, accr, acci):
    i = pl.program_id(0)
    nsteps = pl.num_programs(0)
    bsz = xri_ref.shape[0]

    xr = xri_ref[:, :NF]
    xi = xri_ref[:, NF:]
    a2 = jnp.concatenate([xr, xi], axis=0)
    b2m = jnp.concatenate([xi, xr], axis=0)

    y0 = jnp.dot(a2, w1_ref[0], preferred_element_type=jnp.float32)
    y1 = jnp.dot(b2m, w1_ref[1], preferred_element_type=jnp.float32)
    o1r = jax.nn.relu(y0[:bsz] - y1[:bsz] + b1_ref[0])
    o1i = jax.nn.relu(y0[bsz:] + y1[bsz:] + b1_ref[1])
    o2 = jnp.concatenate([o1r, o1i], axis=0)

    u = jnp.dot(o2, w2_ref[0], preferred_element_type=jnp.float32)
    v = jnp.dot(o2, w2_ref[1], preferred_element_type=jnp.float32)

    @pl.when(i == 0)
    def _init():
        accr[...] = jnp.zeros_like(accr)
        acci[...] = jnp.zeros_like(acci)

    accr[...] += u[:bsz] - v[bsz:]
    acci[...] += u[bsz:] + v[:bsz]

    @pl.when(i == nsteps - 1)
    def _fin():
        ar = accr[...] + b2_ref[0]
        ai = acci[...] + b2_ref[1]
        mag = jnp.sqrt(ar * ar + ai * ai)
        clean = jnp.dot(mag, wg_ref[...], preferred_element_type=jnp.float32)
        zn = jnp.dot(mag, wn_ref[...], preferred_element_type=jnp.float32)
        softplus = jnp.log1p(jnp.exp(-jnp.abs(zn))) + jnp.maximum(zn, 0.0)
        noisy = clean + softplus + eps_ref[0, 0]
        logits = jnp.where(tr_ref[0, 0] != 0.0, noisy, clean)

        col = jax.lax.broadcasted_iota(jnp.int32, logits.shape, 1)
        m1 = jnp.max(logits, axis=1, keepdims=True)
        i1 = jnp.min(jnp.where(logits == m1, col, NPS), axis=1, keepdims=True)
        l2 = jnp.where(col == i1, -jnp.inf, logits)
        m2 = jnp.max(l2, axis=1, keepdims=True)
        i2 = jnp.min(jnp.where(l2 == m2, col, NPS), axis=1, keepdims=True)
        e2 = jnp.exp(m2 - m1)
        denom = 1.0 + e2
        p1 = 1.0 / denom
        p2 = e2 / denom
        o_ref[...] = (jnp.where(col == i1, p1, 0.0)
                      + jnp.where(col == i2, p2, 0.0))


def kernel(x, start_w, start_b, w1, b1, w2, b2, w_gate, w_noise,
           training=False, noise_epsilon=0.01):
    B_, C_, L_, N_ = x.shape
    CN = C_ * N_
    xt = x.transpose(0, 1, 3, 2).reshape(B_ * CN, L_)
    wext = jnp.repeat(start_w[0] / N_, N_).reshape(1, CN)

    xs = pl.pallas_call(
        _front_body,
        grid=(B_,),
        in_specs=[
            pl.BlockSpec((CN, L_), lambda i: (i, 0)),
            pl.BlockSpec((1, CN), lambda i: (0, 0)),
        ],
        out_specs=pl.BlockSpec((1, 1, L_), lambda i: (i, 0, 0)),
        out_shape=jax.ShapeDtypeStruct((B_, 1, L_), jnp.float32),
    )(xt, wext)
    xs = xs.reshape(B_, L_)

    d2 = jnp.asarray(_D2_NP)
    xri = pl.pallas_call(
        _dft_body,
        in_specs=[pl.BlockSpec((B_, SEQ), lambda: (0, 0)),
                  pl.BlockSpec((SEQ, 2 * NF), lambda: (0, 0))],
        out_specs=pl.BlockSpec((B_, 2 * NF), lambda: (0, 0)),
        out_shape=jax.ShapeDtypeStruct((B_, 2 * NF), jnp.float32),
    )(xs, d2)

    tr = jnp.asarray(training, jnp.float32).reshape(1, 1)
    eps = jnp.asarray(noise_epsilon, jnp.float32).reshape(1, 1)
    nh = HID // HBLK

    gates = pl.pallas_call(
        _mlp_body,
        grid=(nh,),
        in_specs=[
            pl.BlockSpec((B_, 2 * NF), lambda i: (0, 0)),
            pl.BlockSpec((2, NF, HBLK), lambda i: (0, 0, i)),
            pl.BlockSpec((2, HBLK), lambda i: (0, i)),
            pl.BlockSpec((2, HBLK, NF), lambda i: (0, i, 0)),
            pl.BlockSpec((2, NF), lambda i: (0, 0)),
            pl.BlockSpec((NF, NPS), lambda i: (0, 0)),
            pl.BlockSpec((NF, NPS), lambda i: (0, 0)),
            pl.BlockSpec((1, 1), lambda i: (0, 0),
                         memory_space=pltpu.SMEM),
            pl.BlockSpec((1, 1), lambda i: (0, 0),
                         memory_space=pltpu.SMEM),
        ],
        out_specs=pl.BlockSpec((B_, NPS), lambda i: (0, 0)),
        out_shape=jax.ShapeDtypeStruct((B_, NPS), jnp.float32),
        scratch_shapes=[pltpu.VMEM((B_, NF), jnp.float32),
                        pltpu.VMEM((B_, NF), jnp.float32)],
    )(xri, w1, b1, w2, b2, w_gate, w_noise, tr, eps)
    return gates


# speedup vs baseline: 6.9477x; 1.0452x over previous
import numpy as np
import jax
import jax.numpy as jnp
from jax.experimental import pallas as pl
from jax.experimental.pallas import tpu as pltpu

SEQ = 2048
NF = SEQ // 2
HID = 4096
NPS = 88
HBLK = 1024

_l = np.arange(SEQ, dtype=np.float64)[:, None]
_k = np.arange(1, NF + 1, dtype=np.float64)[None, :]
_ang = (2.0 * np.pi / SEQ) * _l * _k
_D2_NP = (np.concatenate([np.cos(_ang), -np.sin(_ang)], axis=1)
          / np.sqrt(SEQ)).astype(np.float32)


def _front_body(xt_ref, wext_ref, d2_ref, o_ref, xs_s):
    i = pl.program_id(0)
    nb = pl.num_programs(0)
    xs_s[pl.ds(i, 1), :] = jnp.dot(wext_ref[...], xt_ref[...],
                                   preferred_element_type=jnp.float32)

    @pl.when(i == nb - 1)
    def _fin():
        o_ref[...] = jnp.dot(xs_s[...], d2_ref[...],
                             preferred_element_type=jnp.float32)


def _mlp_body(xri_ref, w1_ref, b1_ref, w2_ref, b2_ref, wg_ref, wn_ref,
              tr_ref, eps_ref, o_ref, accr, acci):
    i = pl.program_id(0)
    nsteps = pl.num_programs(0)
    bsz = xri_ref.shape[0]

    xr = xri_ref[:, :NF]
    xi = xri_ref[:, NF:]
    a2 = jnp.concatenate([xr, xi], axis=0)
    b2m = jnp.concatenate([xi, xr], axis=0)

    y0 = jnp.dot(a2, w1_ref[0], preferred_element_type=jnp.float32)
    y1 = jnp.dot(b2m, w1_ref[1], preferred_element_type=jnp.float32)
    o1r = jax.nn.relu(y0[:bsz] - y1[:bsz] + b1_ref[0])
    o1i = jax.nn.relu(y0[bsz:] + y1[bsz:] + b1_ref[1])
    o2 = jnp.concatenate([o1r, o1i], axis=0)

    u = jnp.dot(o2, w2_ref[0], preferred_element_type=jnp.float32)
    v = jnp.dot(o2, w2_ref[1], preferred_element_type=jnp.float32)

    @pl.when(i == 0)
    def _init():
        accr[...] = jnp.zeros_like(accr)
        acci[...] = jnp.zeros_like(acci)

    accr[...] += u[:bsz] - v[bsz:]
    acci[...] += u[bsz:] + v[:bsz]

    @pl.when(i == nsteps - 1)
    def _fin():
        ar = accr[...] + b2_ref[0]
        ai = acci[...] + b2_ref[1]
        mag = jnp.sqrt(ar * ar + ai * ai)
        clean = jnp.dot(mag, wg_ref[...], preferred_element_type=jnp.float32)
        zn = jnp.dot(mag, wn_ref[...], preferred_element_type=jnp.float32)
        softplus = jnp.log1p(jnp.exp(-jnp.abs(zn))) + jnp.maximum(zn, 0.0)
        noisy = clean + softplus + eps_ref[0, 0]
        logits = jnp.where(tr_ref[0, 0] != 0.0, noisy, clean)

        col = jax.lax.broadcasted_iota(jnp.int32, logits.shape, 1)
        m1 = jnp.max(logits, axis=1, keepdims=True)
        i1 = jnp.min(jnp.where(logits == m1, col, NPS), axis=1, keepdims=True)
        l2 = jnp.where(col == i1, -jnp.inf, logits)
        m2 = jnp.max(l2, axis=1, keepdims=True)
        i2 = jnp.min(jnp.where(l2 == m2, col, NPS), axis=1, keepdims=True)
        e2 = jnp.exp(m2 - m1)
        denom = 1.0 + e2
        p1 = 1.0 / denom
        p2 = e2 / denom
        o_ref[...] = (jnp.where(col == i1, p1, 0.0)
                      + jnp.where(col == i2, p2, 0.0))


def kernel(x, start_w, start_b, w1, b1, w2, b2, w_gate, w_noise,
           training=False, noise_epsilon=0.01):
    B_, C_, L_, N_ = x.shape
    CN = C_ * N_
    xt = x.transpose(0, 1, 3, 2).reshape(B_ * CN, L_)
    wext = jnp.repeat(start_w[0] / N_, N_).reshape(1, CN)

    d2 = jnp.asarray(_D2_NP)
    xri = pl.pallas_call(
        _front_body,
        grid=(B_,),
        in_specs=[
            pl.BlockSpec((CN, L_), lambda i: (i, 0)),
            pl.BlockSpec((1, CN), lambda i: (0, 0)),
            pl.BlockSpec((SEQ, 2 * NF), lambda i: (0, 0)),
        ],
        out_specs=pl.BlockSpec((B_, 2 * NF), lambda i: (0, 0)),
        out_shape=jax.ShapeDtypeStruct((B_, 2 * NF), jnp.float32),
        scratch_shapes=[pltpu.VMEM((B_, SEQ), jnp.float32)],
    )(xt, wext, d2)

    tr = jnp.asarray(training, jnp.float32).reshape(1, 1)
    eps = jnp.asarray(noise_epsilon, jnp.float32).reshape(1, 1)
    nh = HID // HBLK

    gates = pl.pallas_call(
        _mlp_body,
        grid=(nh,),
        in_specs=[
            pl.BlockSpec((B_, 2 * NF), lambda i: (0, 0)),
            pl.BlockSpec((2, NF, HBLK), lambda i: (0, 0, i)),
            pl.BlockSpec((2, HBLK), lambda i: (0, i)),
            pl.BlockSpec((2, HBLK, NF), lambda i: (0, i, 0)),
            pl.BlockSpec((2, NF), lambda i: (0, 0)),
            pl.BlockSpec((NF, NPS), lambda i: (0, 0)),
            pl.BlockSpec((NF, NPS), lambda i: (0, 0)),
            pl.BlockSpec((1, 1), lambda i: (0, 0),
                         memory_space=pltpu.SMEM),
            pl.BlockSpec((1, 1), lambda i: (0, 0),
                         memory_space=pltpu.SMEM),
        ],
        out_specs=pl.BlockSpec((B_, NPS), lambda i: (0, 0)),
        out_shape=jax.ShapeDtypeStruct((B_, NPS), jnp.float32),
        scratch_shapes=[pltpu.VMEM((B_, NF), jnp.float32),
                        pltpu.VMEM((B_, NF), jnp.float32)],
    )(xri, w1, b1, w2, b2, w_gate, w_noise, tr, eps)
    return gates
